# baseline (speedup 1.0000x reference)
import functools

import jax
import jax.numpy as jnp
from jax import lax
from jax.experimental import pallas as pl
from jax.experimental.pallas import tpu as pltpu
from jax.experimental.pallas import tpu_sc as plsc

_NC = 2
_NS = 16
_NW = _NC * _NS
_EB = 80


_NBUF = 3
_PACK_SHIFT = 14


@functools.lru_cache(maxsize=None)
def _make_sc_agg(n_pad, d, nb):
    rows_per_tile = n_pad // _NS
    mesh = plsc.VectorSubcoreMesh(core_axis_name="c", subcore_axis_name="s",
                                  num_cores=_NC, num_subcores=_NS)

    scratch = [
        pltpu.VMEM((nb, _EB), jnp.int32),
        [pltpu.VMEM((_EB,), jnp.int32) for _ in range(_NBUF)],
        [pltpu.VMEM((_EB,), jnp.int32) for _ in range(_NBUF)],
        [pltpu.VMEM((_EB, d), jnp.float32) for _ in range(_NBUF)],
        [pltpu.SemaphoreType.DMA for _ in range(_NBUF)],
        [pltpu.SemaphoreType.DMA for _ in range(_NBUF)],
        pltpu.VMEM_SHARED((n_pad, d), jnp.float32),
    ]

    @functools.partial(
        pl.kernel,
        out_type=jax.ShapeDtypeStruct((_NC, n_pad, d), jnp.float32),
        mesh=mesh,
        scratch_types=scratch,
    )
    def sc_agg(h_hbm, pk_hbm, out_hbm,
               pk_v, srcs, dsts, rows, gsem, ssem, acc_sh):
        c = lax.axis_index("c")
        s = lax.axis_index("s")
        wid = c * _NS + s
        row0 = s * rows_per_tile

        pltpu.sync_copy(pk_hbm.at[wid], pk_v)

        def fire(i, b):
            for j in range(_EB // 16):
                p = pk_v[i, pl.ds(j * 16, 16)]
                srcs[b][pl.ds(j * 16, 16)] = lax.bitwise_and(
                    p, jnp.int32((1 << _PACK_SHIFT) - 1))
                dsts[b][pl.ds(j * 16, 16)] = lax.shift_right_logical(
                    p, jnp.int32(_PACK_SHIFT))
            pltpu.async_copy(h_hbm.at[srcs[b]], rows[b], gsem[b])

        def drain_gather(b):
            pltpu.make_async_copy(h_hbm.at[srcs[b]], rows[b], gsem[b]).wait()

        def drain_scatter(b):
            pltpu.make_async_copy(rows[b], acc_sh.at[dsts[b]],
                                  ssem[b]).wait()

        for b in range(_NBUF - 1):
            fire(b, b)

        zrow = rows[_NBUF - 1]
        zv = jnp.zeros((16,), jnp.float32)

        @pl.loop(0, _EB)
        def _zfill(r):
            for j in range(d // 16):
                zrow[r, pl.ds(j * 16, 16)] = zv

        @pl.loop(0, rows_per_tile // _EB)
        def _zcopy(k):
            pltpu.sync_copy(zrow, acc_sh.at[pl.ds(row0 + k * _EB, _EB)])

        rem = rows_per_tile % _EB
        if rem:
            pltpu.sync_copy(zrow.at[pl.ds(0, rem)],
                            acc_sh.at[pl.ds(row0 + (rows_per_tile // _EB) * _EB,
                                            rem)])
        plsc.subcore_barrier()

        @pl.loop(0, nb // _NBUF)
        def _edge_group(j):
            i0 = j * _NBUF
            for b in range(_NBUF):
                i = i0 + b
                nxt = i + _NBUF - 1
                slot = (b + _NBUF - 1) % _NBUF

                @pl.when(nxt < nb)
                def _():
                    @pl.when(nxt >= _NBUF)
                    def _():
                        drain_scatter(slot)

                    fire(nxt, slot)

                drain_gather(b)
                pltpu.async_copy(rows[b], acc_sh.at[dsts[b]], ssem[b],
                                 add=True)

        for b in range(_NBUF):
            drain_scatter(b)

        plsc.subcore_barrier()
        pltpu.sync_copy(acc_sh.at[pl.ds(row0, rows_per_tile)],
                        out_hbm.at[c, pl.ds(row0, rows_per_tile)])

    return sc_agg


def _sc_agg(h, packed2, n_pad):
    d = h.shape[1]
    nb = packed2.shape[1]
    return _make_sc_agg(n_pad, d, nb)(h, packed2)


def _gin_dense(h, agg2, w1, b1, w2, b2, eps, first):
    n, d = h.shape
    hd = w1.shape[1]
    blk = 1024
    grid = (pl.cdiv(n, blk),)

    def body(h_ref, a_ref, w1_ref, b1_ref, w2_ref, b2_ref, eps_ref, o_ref):
        z = (1.0 + eps_ref[0]) * h_ref[...] + a_ref[0] + a_ref[1]
        a1 = jnp.dot(z, w1_ref[...], preferred_element_type=jnp.float32)
        a1 = jnp.maximum(a1 + b1_ref[...], 0.0)
        o = jnp.dot(a1, w2_ref[...], preferred_element_type=jnp.float32)
        r = jnp.maximum(o + b2_ref[...], 0.0)
        o_ref[...] = r if first else h_ref[...] + r

    return pl.pallas_call(
        body,
        grid=grid,
        in_specs=[
            pl.BlockSpec((blk, d), lambda i: (i, 0)),
            pl.BlockSpec((_NC, blk, d), lambda i: (0, i, 0)),
            pl.BlockSpec((d, hd), lambda i: (0, 0)),
            pl.BlockSpec((1, hd), lambda i: (0, 0)),
            pl.BlockSpec((hd, hd), lambda i: (0, 0)),
            pl.BlockSpec((1, hd), lambda i: (0, 0)),
            pl.BlockSpec(memory_space=pltpu.SMEM),
        ],
        out_specs=pl.BlockSpec((blk, hd), lambda i: (i, 0)),
        out_shape=jax.ShapeDtypeStruct((n, hd), jnp.float32),
    )(h, agg2, w1, b1.reshape(1, -1), w2, b2.reshape(1, -1), eps.reshape(1))


def _sigmoid(x):
    return 1.0 / (1.0 + jnp.exp(-x))


def _ln(x, g, b):
    mu = jnp.mean(x, axis=-1, keepdims=True)
    var = jnp.mean((x - mu) * (x - mu), axis=-1, keepdims=True)
    return (x - mu) / jnp.sqrt(var + 1e-5) * g + b


def _heads(h, agg2, cp, att_p, ang_p, rad_p):
    n, d = h.shape
    blk = 1024
    grid = (pl.cdiv(n, blk),)

    def pad_w(w, out_w=128):
        return jnp.pad(w, ((0, 0), (0, out_w - w.shape[1])))

    def pad_b(b, out_w=128):
        return jnp.pad(b, (0, out_w - b.shape[0])).reshape(1, -1)

    wa = pad_w(att_p['W'])
    ba = pad_b(att_p['b'])
    a1, ab1 = ang_p['lin1']['W'], ang_p['lin1']['b'].reshape(1, -1)
    g1, gb1 = ang_p['ln1']['g'].reshape(1, -1), ang_p['ln1']['b'].reshape(1, -1)
    a2, ab2 = ang_p['lin2']['W'], ang_p['lin2']['b'].reshape(1, -1)
    g2, gb2 = ang_p['ln2']['g'].reshape(1, -1), ang_p['ln2']['b'].reshape(1, -1)
    a3 = pad_w(ang_p['lin3']['W'])
    ab3 = pad_b(ang_p['lin3']['b'])
    r1, rb1 = rad_p['lin1']['W'], rad_p['lin1']['b'].reshape(1, -1)
    rg1, rgb1 = rad_p['ln1']['g'].reshape(1, -1), rad_p['ln1']['b'].reshape(1, -1)
    r2 = pad_w(rad_p['lin2']['W'])
    rb2 = pad_b(rad_p['lin2']['b'])

    d2 = a1.shape[1]

    def body(h_ref, ag_ref, w1_ref, b1_ref, w2_ref, b2_ref, eps_ref,
             wa_r, ba_r, a1_r, ab1_r, g1_r, gb1_r, a2_r, ab2_r,
             g2_r, gb2_r, a3_r, ab3_r, r1_r, rb1_r, rg1_r, rgb1_r,
             r2_r, rb2_r, o_ref):
        z = (1.0 + eps_ref[0]) * h_ref[...] + ag_ref[0] + ag_ref[1]
        t4 = jnp.dot(z, w1_ref[...], preferred_element_type=jnp.float32)
        t4 = jnp.maximum(t4 + b1_ref[...], 0.0)
        o4 = jnp.dot(t4, w2_ref[...], preferred_element_type=jnp.float32)
        hv = h_ref[...] + jnp.maximum(o4 + b2_ref[...], 0.0)
        att = _sigmoid(jnp.dot(hv, wa_r[...],
                               preferred_element_type=jnp.float32) + ba_r[...])
        hw = hv * att[:, 0:1]

        t = jnp.dot(hw, a1_r[...], preferred_element_type=jnp.float32) + ab1_r[...]
        t = jnp.maximum(_ln(t, g1_r[...], gb1_r[...]), 0.0)
        t = jnp.dot(t, a2_r[...], preferred_element_type=jnp.float32) + ab2_r[...]
        t = jnp.maximum(_ln(t, g2_r[...], gb2_r[...]), 0.0)
        ang = jnp.dot(t, a3_r[...], preferred_element_type=jnp.float32) + ab3_r[...]
        theta = ang[:, 0:1]
        phi = ang[:, 1:2]

        u = jnp.dot(hw, r1_r[...], preferred_element_type=jnp.float32) + rb1_r[...]
        u = jnp.maximum(_ln(u, rg1_r[...], rgb1_r[...]), 0.0)
        rr = _sigmoid(jnp.dot(u, r2_r[...],
                              preferred_element_type=jnp.float32) + rb2_r[...])
        r = 0.9 + 0.2 * rr[:, 0:1]

        st = jnp.sin(theta)
        cx = r * st * jnp.cos(phi)
        cy = r * st * jnp.sin(phi)
        cz = r * jnp.cos(theta)
        nrm = jnp.sqrt(cx * cx + cy * cy + cz * cz) + 1e-8
        o_ref[:, 0:1] = cx / nrm
        o_ref[:, 1:2] = cy / nrm
        o_ref[:, 2:3] = cz / nrm

    full = lambda shape: pl.BlockSpec(shape, lambda i: (0, 0))
    return pl.pallas_call(
        body,
        grid=grid,
        in_specs=[
            pl.BlockSpec((blk, d), lambda i: (i, 0)),
            pl.BlockSpec((_NC, blk, d), lambda i: (0, i, 0)),
            full((d, d)), full((1, d)), full((d, d)), full((1, d)),
            pl.BlockSpec(memory_space=pltpu.SMEM),
            full((d, 128)), full((1, 128)),
            full((d, d2)), full((1, d2)), full((1, d2)), full((1, d2)),
            full((d2, d)), full((1, d)), full((1, d)), full((1, d)),
            full((d, 128)), full((1, 128)),
            full((d, d)), full((1, d)), full((1, d)), full((1, d)),
            full((d, 128)), full((1, 128)),
        ],
        out_specs=pl.BlockSpec((blk, 128), lambda i: (i, 0)),
        out_shape=jax.ShapeDtypeStruct((n, 128), jnp.float32),
    )(h, agg2, cp['lin1']['W'], cp['lin1']['b'].reshape(1, -1),
      cp['lin2']['W'], cp['lin2']['b'].reshape(1, -1), cp['eps'].reshape(1),
      wa, ba, a1, ab1, g1, gb1, a2, ab2, g2, gb2, a3, ab3,
      r1, rb1, rg1, rgb1, r2, rb2)


def kernel(x, edge_index, params):
    src = edge_index[0]
    dst = edge_index[1]
    n, d = x.shape
    e = src.shape[0]
    n_pad = ((n + 127) // 128) * 128

    grp = _NW * _EB * _NBUF
    e_pad = ((e + grp - 1) // grp) * grp
    pad_n = e_pad - e
    pad_src = (jnp.arange(pad_n, dtype=jnp.int32) * 97) % n
    pad_dst = n + (jnp.arange(pad_n, dtype=jnp.int32) % (n_pad - n))
    src_p = jnp.concatenate([src, pad_src])
    dst_p = jnp.concatenate([dst, pad_dst])
    packed2 = (src_p | (dst_p << _PACK_SHIFT)).reshape(_NW, -1, _EB)

    h = x
    convs = params['convs']
    for i, cp in enumerate(convs[:-1]):
        agg2 = _sc_agg(h, packed2, n_pad)
        h = _gin_dense(h, agg2, cp['lin1']['W'], cp['lin1']['b'],
                       cp['lin2']['W'], cp['lin2']['b'], cp['eps'],
                       first=(i == 0))

    agg2 = _sc_agg(h, packed2, n_pad)
    coords = _heads(h, agg2, convs[-1], params['attention'],
                    params['angle'], params['radius'])
    return coords[:, :3]


# speedup vs baseline: 1.0180x; 1.0180x over previous
import functools

import jax
import jax.numpy as jnp
from jax import lax
from jax.experimental import pallas as pl
from jax.experimental.pallas import tpu as pltpu
from jax.experimental.pallas import tpu_sc as plsc

_NC = 2
_NS = 16
_NW = _NC * _NS
_EB = 80


_NBUF = 3
_PACK_SHIFT = 14


@functools.lru_cache(maxsize=None)
def _make_sc_agg(n_pad, d, nb):
    rows_per_tile = n_pad // _NS
    mesh = plsc.VectorSubcoreMesh(core_axis_name="c", subcore_axis_name="s",
                                  num_cores=_NC, num_subcores=_NS)

    scratch = [
        pltpu.VMEM((nb, _EB), jnp.int32),
        [pltpu.VMEM((_EB,), jnp.int32) for _ in range(_NBUF)],
        [pltpu.VMEM((_EB,), jnp.int32) for _ in range(_NBUF)],
        [pltpu.VMEM((_EB, d), jnp.float32) for _ in range(_NBUF)],
        [pltpu.SemaphoreType.DMA for _ in range(_NBUF)],
        [pltpu.SemaphoreType.DMA for _ in range(_NBUF)],
        pltpu.VMEM_SHARED((n_pad, d), jnp.float32),
    ]

    @functools.partial(
        pl.kernel,
        out_type=jax.ShapeDtypeStruct((_NC, n_pad, d), jnp.float32),
        mesh=mesh,
        scratch_types=scratch,
    )
    def sc_agg(h_hbm, pk_hbm, out_hbm,
               pk_v, srcs, dsts, rows, gsem, ssem, acc_sh):
        c = lax.axis_index("c")
        s = lax.axis_index("s")
        wid = c * _NS + s
        row0 = s * rows_per_tile

        pltpu.sync_copy(pk_hbm.at[wid], pk_v)

        def fire(i, b):
            for j in range(_EB // 16):
                p = pk_v[i, pl.ds(j * 16, 16)]
                srcs[b][pl.ds(j * 16, 16)] = lax.bitwise_and(
                    p, jnp.int32((1 << _PACK_SHIFT) - 1))
                dsts[b][pl.ds(j * 16, 16)] = lax.shift_right_logical(
                    p, jnp.int32(_PACK_SHIFT))
            pltpu.async_copy(h_hbm.at[srcs[b]], rows[b], gsem[b])

        def drain_gather(b):
            pltpu.make_async_copy(h_hbm.at[srcs[b]], rows[b], gsem[b]).wait()

        def drain_scatter(b):
            pltpu.make_async_copy(rows[b], acc_sh.at[dsts[b]],
                                  ssem[b]).wait()

        for b in range(_NBUF - 1):
            fire(b, b)

        zrow = rows[_NBUF - 1]
        zv = jnp.zeros((16,), jnp.float32)

        @pl.loop(0, _EB)
        def _zfill(r):
            for j in range(d // 16):
                zrow[r, pl.ds(j * 16, 16)] = zv

        @pl.loop(0, rows_per_tile // _EB)
        def _zcopy(k):
            pltpu.sync_copy(zrow, acc_sh.at[pl.ds(row0 + k * _EB, _EB)])

        rem = rows_per_tile % _EB
        if rem:
            pltpu.sync_copy(zrow.at[pl.ds(0, rem)],
                            acc_sh.at[pl.ds(row0 + (rows_per_tile // _EB) * _EB,
                                            rem)])
        plsc.subcore_barrier()

        @pl.loop(0, nb // _NBUF)
        def _edge_group(j):
            i0 = j * _NBUF
            for b in range(_NBUF):
                i = i0 + b
                nxt = i + _NBUF - 1
                slot = (b + _NBUF - 1) % _NBUF

                @pl.when(nxt < nb)
                def _():
                    @pl.when(nxt >= _NBUF)
                    def _():
                        drain_scatter(slot)

                    fire(nxt, slot)

                drain_gather(b)
                pltpu.async_copy(rows[b], acc_sh.at[dsts[b]], ssem[b],
                                 add=True)

        for b in range(_NBUF):
            drain_scatter(b)

        plsc.subcore_barrier()
        pltpu.sync_copy(acc_sh.at[pl.ds(row0, rows_per_tile)],
                        out_hbm.at[c, pl.ds(row0, rows_per_tile)])

    return sc_agg


def _sc_agg(h, packed2, n_pad):
    d = h.shape[1]
    nb = packed2.shape[1]
    return _make_sc_agg(n_pad, d, nb)(h, packed2)


def _gin_dense(h, agg2, w1, b1, w2, b2, eps, first):
    n, d = h.shape
    hd = w1.shape[1]
    blk = 2048
    grid = (pl.cdiv(n, blk),)

    def body(h_ref, a_ref, w1_ref, b1_ref, w2_ref, b2_ref, eps_ref, o_ref):
        z = (1.0 + eps_ref[0]) * h_ref[...] + a_ref[0] + a_ref[1]
        a1 = jnp.dot(z, w1_ref[...], preferred_element_type=jnp.float32)
        a1 = jnp.maximum(a1 + b1_ref[...], 0.0)
        o = jnp.dot(a1, w2_ref[...], preferred_element_type=jnp.float32)
        r = jnp.maximum(o + b2_ref[...], 0.0)
        o_ref[...] = r if first else h_ref[...] + r

    return pl.pallas_call(
        body,
        grid=grid,
        in_specs=[
            pl.BlockSpec((blk, d), lambda i: (i, 0)),
            pl.BlockSpec((_NC, blk, d), lambda i: (0, i, 0)),
            pl.BlockSpec((d, hd), lambda i: (0, 0)),
            pl.BlockSpec((1, hd), lambda i: (0, 0)),
            pl.BlockSpec((hd, hd), lambda i: (0, 0)),
            pl.BlockSpec((1, hd), lambda i: (0, 0)),
            pl.BlockSpec(memory_space=pltpu.SMEM),
        ],
        out_specs=pl.BlockSpec((blk, hd), lambda i: (i, 0)),
        out_shape=jax.ShapeDtypeStruct((n, hd), jnp.float32),
    )(h, agg2, w1, b1.reshape(1, -1), w2, b2.reshape(1, -1), eps.reshape(1))


def _sigmoid(x):
    return 1.0 / (1.0 + jnp.exp(-x))


def _ln(x, g, b):
    mu = jnp.mean(x, axis=-1, keepdims=True)
    var = jnp.mean((x - mu) * (x - mu), axis=-1, keepdims=True)
    return (x - mu) / jnp.sqrt(var + 1e-5) * g + b


def _heads(h, agg2, cp, att_p, ang_p, rad_p):
    n, d = h.shape
    blk = 2048
    grid = (pl.cdiv(n, blk),)

    def pad_w(w, out_w=128):
        return jnp.pad(w, ((0, 0), (0, out_w - w.shape[1])))

    def pad_b(b, out_w=128):
        return jnp.pad(b, (0, out_w - b.shape[0])).reshape(1, -1)

    wa = pad_w(att_p['W'])
    ba = pad_b(att_p['b'])
    a1, ab1 = ang_p['lin1']['W'], ang_p['lin1']['b'].reshape(1, -1)
    g1, gb1 = ang_p['ln1']['g'].reshape(1, -1), ang_p['ln1']['b'].reshape(1, -1)
    a2, ab2 = ang_p['lin2']['W'], ang_p['lin2']['b'].reshape(1, -1)
    g2, gb2 = ang_p['ln2']['g'].reshape(1, -1), ang_p['ln2']['b'].reshape(1, -1)
    a3 = pad_w(ang_p['lin3']['W'])
    ab3 = pad_b(ang_p['lin3']['b'])
    r1, rb1 = rad_p['lin1']['W'], rad_p['lin1']['b'].reshape(1, -1)
    rg1, rgb1 = rad_p['ln1']['g'].reshape(1, -1), rad_p['ln1']['b'].reshape(1, -1)
    r2 = pad_w(rad_p['lin2']['W'])
    rb2 = pad_b(rad_p['lin2']['b'])

    d2 = a1.shape[1]

    def body(h_ref, ag_ref, w1_ref, b1_ref, w2_ref, b2_ref, eps_ref,
             wa_r, ba_r, a1_r, ab1_r, g1_r, gb1_r, a2_r, ab2_r,
             g2_r, gb2_r, a3_r, ab3_r, r1_r, rb1_r, rg1_r, rgb1_r,
             r2_r, rb2_r, o_ref):
        z = (1.0 + eps_ref[0]) * h_ref[...] + ag_ref[0] + ag_ref[1]
        t4 = jnp.dot(z, w1_ref[...], preferred_element_type=jnp.float32)
        t4 = jnp.maximum(t4 + b1_ref[...], 0.0)
        o4 = jnp.dot(t4, w2_ref[...], preferred_element_type=jnp.float32)
        hv = h_ref[...] + jnp.maximum(o4 + b2_ref[...], 0.0)
        att = _sigmoid(jnp.dot(hv, wa_r[...],
                               preferred_element_type=jnp.float32) + ba_r[...])
        hw = hv * att[:, 0:1]

        t = jnp.dot(hw, a1_r[...], preferred_element_type=jnp.float32) + ab1_r[...]
        t = jnp.maximum(_ln(t, g1_r[...], gb1_r[...]), 0.0)
        t = jnp.dot(t, a2_r[...], preferred_element_type=jnp.float32) + ab2_r[...]
        t = jnp.maximum(_ln(t, g2_r[...], gb2_r[...]), 0.0)
        ang = jnp.dot(t, a3_r[...], preferred_element_type=jnp.float32) + ab3_r[...]
        theta = ang[:, 0:1]
        phi = ang[:, 1:2]

        u = jnp.dot(hw, r1_r[...], preferred_element_type=jnp.float32) + rb1_r[...]
        u = jnp.maximum(_ln(u, rg1_r[...], rgb1_r[...]), 0.0)
        rr = _sigmoid(jnp.dot(u, r2_r[...],
                              preferred_element_type=jnp.float32) + rb2_r[...])
        r = 0.9 + 0.2 * rr[:, 0:1]

        st = jnp.sin(theta)
        cx = r * st * jnp.cos(phi)
        cy = r * st * jnp.sin(phi)
        cz = r * jnp.cos(theta)
        nrm = jnp.sqrt(cx * cx + cy * cy + cz * cz) + 1e-8
        o_ref[:, 0:1] = cx / nrm
        o_ref[:, 1:2] = cy / nrm
        o_ref[:, 2:3] = cz / nrm

    full = lambda shape: pl.BlockSpec(shape, lambda i: (0, 0))
    return pl.pallas_call(
        body,
        grid=grid,
        in_specs=[
            pl.BlockSpec((blk, d), lambda i: (i, 0)),
            pl.BlockSpec((_NC, blk, d), lambda i: (0, i, 0)),
            full((d, d)), full((1, d)), full((d, d)), full((1, d)),
            pl.BlockSpec(memory_space=pltpu.SMEM),
            full((d, 128)), full((1, 128)),
            full((d, d2)), full((1, d2)), full((1, d2)), full((1, d2)),
            full((d2, d)), full((1, d)), full((1, d)), full((1, d)),
            full((d, 128)), full((1, 128)),
            full((d, d)), full((1, d)), full((1, d)), full((1, d)),
            full((d, 128)), full((1, 128)),
        ],
        out_specs=pl.BlockSpec((blk, 128), lambda i: (i, 0)),
        out_shape=jax.ShapeDtypeStruct((n, 128), jnp.float32),
    )(h, agg2, cp['lin1']['W'], cp['lin1']['b'].reshape(1, -1),
      cp['lin2']['W'], cp['lin2']['b'].reshape(1, -1), cp['eps'].reshape(1),
      wa, ba, a1, ab1, g1, gb1, a2, ab2, g2, gb2, a3, ab3,
      r1, rb1, rg1, rgb1, r2, rb2)


def kernel(x, edge_index, params):
    src = edge_index[0]
    dst = edge_index[1]
    n, d = x.shape
    e = src.shape[0]
    n_pad = ((n + 127) // 128) * 128

    grp = _NW * _EB * _NBUF
    e_pad = ((e + grp - 1) // grp) * grp
    pad_n = e_pad - e
    pad_src = (jnp.arange(pad_n, dtype=jnp.int32) * 97) % n
    pad_dst = n + (jnp.arange(pad_n, dtype=jnp.int32) % (n_pad - n))
    src_p = jnp.concatenate([src, pad_src])
    dst_p = jnp.concatenate([dst, pad_dst])
    packed2 = (src_p | (dst_p << _PACK_SHIFT)).reshape(_NW, -1, _EB)

    h = x
    convs = params['convs']
    for i, cp in enumerate(convs[:-1]):
        agg2 = _sc_agg(h, packed2, n_pad)
        h = _gin_dense(h, agg2, cp['lin1']['W'], cp['lin1']['b'],
                       cp['lin2']['W'], cp['lin2']['b'], cp['eps'],
                       first=(i == 0))

    agg2 = _sc_agg(h, packed2, n_pad)
    coords = _heads(h, agg2, convs[-1], params['attention'],
                    params['angle'], params['radius'])
    return coords[:, :3]
